# Initial kernel scaffold; baseline (speedup 1.0000x reference)
import jax
import jax.numpy as jnp
from jax.experimental import pallas as pl


def kernel(x, thresholds):
    raise NotImplementedError("write your pallas kernel here")


# speedup vs baseline: 35.4906x; 35.4906x over previous
import functools

import jax
import jax.numpy as jnp
from jax.experimental import pallas as pl
from jax.experimental.pallas import tpu as pltpu

_SPARSITY = 0.5
_MOMENTUM = 0.1
_NIT = 16
_C_CHUNK = 128


def _select_body(x_ref, t_ref, o_ref, bits_ref, p_ref, *, k):
    j = pl.program_id(1)

    @pl.when(j == 0)
    def _init():
        bits_ref[...] = (
            jax.lax.bitcast_convert_type(x_ref[...], jnp.int32)
            & jnp.int32(0x7FFFFFFF)
        )
        p_ref[...] = jnp.zeros_like(p_ref)

    @pl.when(j < _NIT)
    def _search():
        bit = jnp.int32(1) << (30 - j)
        cand = p_ref[...] | bit
        cmp = (bits_ref[...] < cand[None, :, None]).astype(jnp.int32)
        cnt = jnp.sum(cmp, axis=(0, 2))
        p_ref[...] = jnp.where(cnt >= k, p_ref[...], cand)

    @pl.when(j == _NIT)
    def _finalize():
        est_bits = p_ref[...] + (jnp.int32(1) << (30 - _NIT))
        kth = jax.lax.bitcast_convert_type(est_bits, jnp.float32)
        thr = t_ref[...] * (1.0 - _MOMENTUM) + kth * _MOMENTUM
        xv = x_ref[...]
        o_ref[...] = jnp.where(jnp.abs(xv) > thr[None, :, None], xv, 0.0)


def kernel(x, thresholds):
    N, C, L = x.shape
    k = max(1, int(N * L * _SPARSITY))
    grid = (C // _C_CHUNK, _NIT + 1)
    out = pl.pallas_call(
        functools.partial(_select_body, k=k),
        grid=grid,
        in_specs=[
            pl.BlockSpec((N, _C_CHUNK, L), lambda i, j: (0, i, 0)),
            pl.BlockSpec((_C_CHUNK,), lambda i, j: (i,)),
        ],
        out_specs=pl.BlockSpec((N, _C_CHUNK, L), lambda i, j: (0, i, 0)),
        out_shape=jax.ShapeDtypeStruct((N, C, L), jnp.float32),
        scratch_shapes=[
            pltpu.VMEM((N, _C_CHUNK, L), jnp.int32),
            pltpu.VMEM((_C_CHUNK,), jnp.int32),
        ],
        compiler_params=pltpu.CompilerParams(
            dimension_semantics=("arbitrary", "arbitrary"),
        ),
    )(x, thresholds)
    return out
